# baseline (speedup 1.0000x reference)
import jax
import jax.numpy as jnp
from jax.experimental import pallas as pl
from jax.experimental.pallas import tpu as pltpu


def _copy_body(v_ref, o_ref, sem):
    cp = pltpu.make_async_copy(v_ref, o_ref, sem)
    cp.start()
    cp.wait()


def kernel(queries, keys, values):
    vt = jnp.transpose(values, (0, 2, 3, 1))
    out = pl.pallas_call(
        _copy_body,
        in_specs=[pl.BlockSpec(memory_space=pltpu.MemorySpace.HBM)],
        out_specs=pl.BlockSpec(memory_space=pltpu.MemorySpace.HBM),
        out_shape=jax.ShapeDtypeStruct(vt.shape, vt.dtype),
        scratch_shapes=[pltpu.SemaphoreType.DMA],
    )(vt)
    return jnp.transpose(out, (0, 1, 3, 2))


# speedup vs baseline: 20.7205x; 20.7205x over previous
import jax
import jax.numpy as jnp
from jax.experimental import pallas as pl


def _copy_body(v_ref, o_ref):
    o_ref[...] = v_ref[...]


def kernel(queries, keys, values):
    b, l, h, d = values.shape
    vt = jnp.transpose(values, (0, 2, 3, 1)).reshape(b * h, d, l)
    out = pl.pallas_call(
        _copy_body,
        grid=(b * h,),
        in_specs=[pl.BlockSpec((1, d, l), lambda i: (i, 0, 0))],
        out_specs=pl.BlockSpec((1, d, l), lambda i: (i, 0, 0)),
        out_shape=jax.ShapeDtypeStruct((b * h, d, l), values.dtype),
    )(vt)
    return jnp.transpose(out.reshape(b, h, d, l), (0, 1, 3, 2))


# speedup vs baseline: 22.2324x; 1.0730x over previous
import jax
import jax.numpy as jnp
from jax.experimental import pallas as pl
from jax.experimental.pallas import tpu as pltpu

_CHUNKS = 24
_NBUF = 8


def _dma_pipeline_body(v_ref, o_ref, buf, in_sems, out_sems):
    def in_copy(k):
        s = k % _NBUF
        return pltpu.make_async_copy(v_ref.at[k], buf.at[s], in_sems.at[s])

    def out_copy(k):
        s = k % _NBUF
        return pltpu.make_async_copy(buf.at[s], o_ref.at[k], out_sems.at[s])

    for k in range(_NBUF):
        in_copy(k).start()
    for k in range(_CHUNKS):
        in_copy(k).wait()
        out_copy(k).start()
        nxt = k + _NBUF
        if nxt < _CHUNKS:
            out_copy(k).wait()
            in_copy(nxt).start()
    for k in range(_CHUNKS - _NBUF, _CHUNKS):
        out_copy(k).wait()


def kernel(queries, keys, values):
    b, l, h, d = values.shape
    vt = jnp.transpose(values, (0, 2, 3, 1)).reshape(_CHUNKS, (b * h * d * l) // (_CHUNKS * l), l)
    out = pl.pallas_call(
        _dma_pipeline_body,
        in_specs=[pl.BlockSpec(memory_space=pltpu.MemorySpace.HBM)],
        out_specs=pl.BlockSpec(memory_space=pltpu.MemorySpace.HBM),
        out_shape=jax.ShapeDtypeStruct(vt.shape, vt.dtype),
        scratch_shapes=[
            pltpu.VMEM((_NBUF,) + vt.shape[1:], vt.dtype),
            pltpu.SemaphoreType.DMA((_NBUF,)),
            pltpu.SemaphoreType.DMA((_NBUF,)),
        ],
    )(vt)
    return jnp.transpose(out.reshape(b, h, d, l), (0, 1, 3, 2))


# speedup vs baseline: 33.6830x; 1.5150x over previous
import jax
import jax.numpy as jnp
from jax.experimental import pallas as pl
from jax.experimental.pallas import tpu as pltpu

_CHUNKS = 12
_NBUF = 6


def _dma_pipeline_body(v_ref, o_ref, buf, in_sems, out_sems):
    def in_copy(k):
        s = k % _NBUF
        return pltpu.make_async_copy(v_ref.at[k], buf.at[s], in_sems.at[s])

    def out_copy(k):
        s = k % _NBUF
        return pltpu.make_async_copy(buf.at[s], o_ref.at[k], out_sems.at[s])

    for k in range(_NBUF):
        in_copy(k).start()
    for k in range(_CHUNKS):
        in_copy(k).wait()
        out_copy(k).start()
        nxt = k + _NBUF
        if nxt < _CHUNKS:
            out_copy(k).wait()
            in_copy(nxt).start()
    for k in range(_CHUNKS - _NBUF, _CHUNKS):
        out_copy(k).wait()


def kernel(queries, keys, values):
    b, l, h, d = values.shape
    vt = jnp.transpose(values, (0, 2, 3, 1)).reshape(_CHUNKS, (b * h * d) // _CHUNKS, l)
    out = pl.pallas_call(
        _dma_pipeline_body,
        in_specs=[pl.BlockSpec(memory_space=pltpu.MemorySpace.HBM)],
        out_specs=pl.BlockSpec(memory_space=pltpu.MemorySpace.HBM),
        out_shape=jax.ShapeDtypeStruct(vt.shape, vt.dtype),
        scratch_shapes=[
            pltpu.VMEM((_NBUF,) + vt.shape[1:], vt.dtype),
            pltpu.SemaphoreType.DMA((_NBUF,)),
            pltpu.SemaphoreType.DMA((_NBUF,)),
        ],
    )(vt)
    return jnp.transpose(out.reshape(b, h, d, l), (0, 1, 3, 2))


# speedup vs baseline: 40.3417x; 1.1977x over previous
import jax
import jax.numpy as jnp
from jax.experimental import pallas as pl
from jax.experimental.pallas import tpu as pltpu

_CHUNKS = 6
_NBUF = 4


def _dma_pipeline_body(v_ref, o_ref, buf, in_sems, out_sems):
    def in_copy(k):
        s = k % _NBUF
        return pltpu.make_async_copy(v_ref.at[k], buf.at[s], in_sems.at[s])

    def out_copy(k):
        s = k % _NBUF
        return pltpu.make_async_copy(buf.at[s], o_ref.at[k], out_sems.at[s])

    for k in range(_NBUF):
        in_copy(k).start()
    for k in range(_CHUNKS):
        in_copy(k).wait()
        out_copy(k).start()
        nxt = k + _NBUF
        if nxt < _CHUNKS:
            out_copy(k).wait()
            in_copy(nxt).start()
    for k in range(_CHUNKS - _NBUF, _CHUNKS):
        out_copy(k).wait()


def kernel(queries, keys, values):
    b, l, h, d = values.shape
    vt = jnp.transpose(values, (0, 2, 3, 1)).reshape(_CHUNKS, (b * h * d) // _CHUNKS, l)
    out = pl.pallas_call(
        _dma_pipeline_body,
        in_specs=[pl.BlockSpec(memory_space=pltpu.MemorySpace.HBM)],
        out_specs=pl.BlockSpec(memory_space=pltpu.MemorySpace.HBM),
        out_shape=jax.ShapeDtypeStruct(vt.shape, vt.dtype),
        scratch_shapes=[
            pltpu.VMEM((_NBUF,) + vt.shape[1:], vt.dtype),
            pltpu.SemaphoreType.DMA((_NBUF,)),
            pltpu.SemaphoreType.DMA((_NBUF,)),
        ],
    )(vt)
    return jnp.transpose(out.reshape(b, h, d, l), (0, 1, 3, 2))


# speedup vs baseline: 44.6498x; 1.1068x over previous
import jax
import jax.numpy as jnp
from jax.experimental import pallas as pl
from jax.experimental.pallas import tpu as pltpu

_CHUNKS = 4
_NBUF = 4


def _dma_pipeline_body(v_ref, o_ref, buf, in_sems, out_sems):
    def in_copy(k):
        s = k % _NBUF
        return pltpu.make_async_copy(v_ref.at[k], buf.at[s], in_sems.at[s])

    def out_copy(k):
        s = k % _NBUF
        return pltpu.make_async_copy(buf.at[s], o_ref.at[k], out_sems.at[s])

    for k in range(_NBUF):
        in_copy(k).start()
    for k in range(_CHUNKS):
        in_copy(k).wait()
        out_copy(k).start()
        nxt = k + _NBUF
        if nxt < _CHUNKS:
            out_copy(k).wait()
            in_copy(nxt).start()
    for k in range(_CHUNKS - _NBUF, _CHUNKS):
        out_copy(k).wait()


def kernel(queries, keys, values):
    b, l, h, d = values.shape
    vt = jnp.transpose(values, (0, 2, 3, 1)).reshape(_CHUNKS, (b * h * d) // _CHUNKS, l)
    out = pl.pallas_call(
        _dma_pipeline_body,
        in_specs=[pl.BlockSpec(memory_space=pltpu.MemorySpace.HBM)],
        out_specs=pl.BlockSpec(memory_space=pltpu.MemorySpace.HBM),
        out_shape=jax.ShapeDtypeStruct(vt.shape, vt.dtype),
        scratch_shapes=[
            pltpu.VMEM((_NBUF,) + vt.shape[1:], vt.dtype),
            pltpu.SemaphoreType.DMA((_NBUF,)),
            pltpu.SemaphoreType.DMA((_NBUF,)),
        ],
    )(vt)
    return jnp.transpose(out.reshape(b, h, d, l), (0, 1, 3, 2))


# speedup vs baseline: 45.1675x; 1.0116x over previous
import jax
import jax.numpy as jnp
from jax.experimental import pallas as pl
from jax.experimental.pallas import tpu as pltpu

_CHUNKS = 3
_NBUF = 3


def _dma_pipeline_body(v_ref, o_ref, buf, in_sems, out_sems):
    def in_copy(k):
        s = k % _NBUF
        return pltpu.make_async_copy(v_ref.at[k], buf.at[s], in_sems.at[s])

    def out_copy(k):
        s = k % _NBUF
        return pltpu.make_async_copy(buf.at[s], o_ref.at[k], out_sems.at[s])

    for k in range(_NBUF):
        in_copy(k).start()
    for k in range(_CHUNKS):
        in_copy(k).wait()
        out_copy(k).start()
        nxt = k + _NBUF
        if nxt < _CHUNKS:
            out_copy(k).wait()
            in_copy(nxt).start()
    for k in range(_CHUNKS - _NBUF, _CHUNKS):
        out_copy(k).wait()


def kernel(queries, keys, values):
    b, l, h, d = values.shape
    vt = jnp.transpose(values, (0, 2, 3, 1)).reshape(_CHUNKS, (b * h * d) // _CHUNKS, l)
    out = pl.pallas_call(
        _dma_pipeline_body,
        in_specs=[pl.BlockSpec(memory_space=pltpu.MemorySpace.HBM)],
        out_specs=pl.BlockSpec(memory_space=pltpu.MemorySpace.HBM),
        out_shape=jax.ShapeDtypeStruct(vt.shape, vt.dtype),
        scratch_shapes=[
            pltpu.VMEM((_NBUF,) + vt.shape[1:], vt.dtype),
            pltpu.SemaphoreType.DMA((_NBUF,)),
            pltpu.SemaphoreType.DMA((_NBUF,)),
        ],
    )(vt)
    return jnp.transpose(out.reshape(b, h, d, l), (0, 1, 3, 2))


# speedup vs baseline: 45.2176x; 1.0011x over previous
import jax
import jax.numpy as jnp
from jax.experimental import pallas as pl
from jax.experimental.pallas import tpu as pltpu

_CHUNKS = 6
_NBUF = 6


def _dma_pipeline_body(v_ref, o_ref, buf, in_sems, out_sems):
    def in_copy(k):
        s = k % _NBUF
        return pltpu.make_async_copy(v_ref.at[k], buf.at[s], in_sems.at[s])

    def out_copy(k):
        s = k % _NBUF
        return pltpu.make_async_copy(buf.at[s], o_ref.at[k], out_sems.at[s])

    for k in range(_NBUF):
        in_copy(k).start()
    for k in range(_CHUNKS):
        in_copy(k).wait()
        out_copy(k).start()
        nxt = k + _NBUF
        if nxt < _CHUNKS:
            out_copy(k).wait()
            in_copy(nxt).start()
    for k in range(_CHUNKS - _NBUF, _CHUNKS):
        out_copy(k).wait()


def kernel(queries, keys, values):
    b, l, h, d = values.shape
    vt = jnp.transpose(values, (0, 2, 3, 1)).reshape(_CHUNKS, (b * h * d) // _CHUNKS, l)
    out = pl.pallas_call(
        _dma_pipeline_body,
        in_specs=[pl.BlockSpec(memory_space=pltpu.MemorySpace.HBM)],
        out_specs=pl.BlockSpec(memory_space=pltpu.MemorySpace.HBM),
        out_shape=jax.ShapeDtypeStruct(vt.shape, vt.dtype),
        scratch_shapes=[
            pltpu.VMEM((_NBUF,) + vt.shape[1:], vt.dtype),
            pltpu.SemaphoreType.DMA((_NBUF,)),
            pltpu.SemaphoreType.DMA((_NBUF,)),
        ],
    )(vt)
    return jnp.transpose(out.reshape(b, h, d, l), (0, 1, 3, 2))
